# Initial kernel scaffold; baseline (speedup 1.0000x reference)
import jax
import jax.numpy as jnp
from jax.experimental import pallas as pl


def kernel(points, knn_idx):
    raise NotImplementedError("write your pallas kernel here")


# speedup vs baseline: 486.7197x; 486.7197x over previous
import functools

import jax
import jax.numpy as jnp
from jax import lax
from jax.experimental import pallas as pl
from jax.experimental.pallas import tpu as pltpu
from jax.experimental.pallas import tpu_sc as plsc

NN_SIZE = 16
RADIUS2 = 0.05 * 0.05

B, N, C = 8, 16384, 3

NC = 2
NS = 16
L = 16
NW = NC * NS
WPB = NW // B
R = N // WPB
GROUPS = R // L


def _rsqrt(x):
    i = plsc.bitcast(x, jnp.int32)
    i = jnp.int32(0x5F3759DF) - lax.shift_right_logical(i, 1)
    y = plsc.bitcast(i, jnp.float32)
    for _ in range(3):
        y = y * (1.5 - 0.5 * x * y * y)
    return y


@functools.partial(
    pl.kernel,
    mesh=plsc.VectorSubcoreMesh(core_axis_name="c", subcore_axis_name="s"),
    compiler_params=pltpu.CompilerParams(needs_layout_passes=False),
    out_type=jax.ShapeDtypeStruct((NW * L,), jnp.float32),
    scratch_types=[
        pltpu.VMEM((N,), jnp.float32),
        pltpu.VMEM((N,), jnp.float32),
        pltpu.VMEM((N,), jnp.float32),
        pltpu.VMEM((NN_SIZE * R,), jnp.int32),
        pltpu.VMEM((L,), jnp.float32),
        pltpu.SemaphoreType.DMA,
    ],
)
def _repulsion_sc(pts_hbm, idx_hbm, out_hbm, x_v, y_v, z_v, idx_v, acc_v, sem):
    wid = lax.axis_index("s") * NC + lax.axis_index("c")
    b = wid // WPB
    base = (wid % WPB) * R

    copies = [
        pltpu.async_copy(pts_hbm.at[pl.ds((b * 3 + 0) * N, N)], x_v, sem),
        pltpu.async_copy(pts_hbm.at[pl.ds((b * 3 + 1) * N, N)], y_v, sem),
        pltpu.async_copy(pts_hbm.at[pl.ds((b * 3 + 2) * N, N)], z_v, sem),
    ]
    for k in range(NN_SIZE):
        copies.append(
            pltpu.async_copy(
                idx_hbm.at[pl.ds((b * NN_SIZE + k) * N + base, R)],
                idx_v.at[pl.ds(k * R, R)],
                sem,
            )
        )
    for c in copies:
        c.wait()

    def body(g, acc):
        gbase = g * L
        cx = x_v[pl.ds(base + gbase, L)]
        cy = y_v[pl.ds(base + gbase, L)]
        cz = z_v[pl.ds(base + gbase, L)]
        for k in range(NN_SIZE):
            nidx = idx_v[pl.ds(k * R + gbase, L)]
            dx = plsc.load_gather(x_v, [nidx]) - cx
            dy = plsc.load_gather(y_v, [nidx]) - cy
            dz = plsc.load_gather(z_v, [nidx]) - cz
            d2 = (dx * dx + dy * dy) + dz * dz
            val = _rsqrt(d2 + 0.0001)
            acc = acc + jnp.where(d2 < RADIUS2, val, 0.0)
        return acc

    acc = lax.fori_loop(0, GROUPS, body, jnp.zeros((L,), jnp.float32))
    acc_v[...] = acc
    pltpu.sync_copy(acc_v, out_hbm.at[pl.ds(wid * L, L)])


def kernel(points, knn_idx):
    pts_t = jnp.swapaxes(points, 1, 2).reshape(B * 3 * N)
    idx_t = jnp.swapaxes(knn_idx, 1, 2).reshape(B * NN_SIZE * N)
    partials = _repulsion_sc(pts_t, idx_t)
    return jnp.sum(partials) / (B * N * NN_SIZE)
